# Initial kernel scaffold; baseline (speedup 1.0000x reference)
import jax
import jax.numpy as jnp
from jax.experimental import pallas as pl


def kernel(x, m_batch, W1a, b1a, W2a, b2a, W1c, b1c, W2c, b2c):
    raise NotImplementedError("write your pallas kernel here")


# speedup vs baseline: 3.4259x; 3.4259x over previous
import functools

import jax
import jax.numpy as jnp
from jax import lax
from jax.experimental import pallas as pl
from jax.experimental.pallas import tpu as pltpu
from jax.experimental.pallas import tpu_sc as plsc

_NUM_SEG = 10000
_N_ROWS = 320000
_D = 128
_NC = 2
_NS = 16
_NW = _NC * _NS
_ROWS_PER_W = _N_ROWS // _NW
_CHUNK = 80
_N_CHUNKS = _ROWS_PER_W // _CHUNK
_SEG_PER_TILE = _NUM_SEG // _NS


def _sc_segment_sum(x, ids):
    mesh = plsc.VectorSubcoreMesh(core_axis_name="c", subcore_axis_name="s")
    zeros = jnp.zeros((_SEG_PER_TILE, _D), jnp.float32)
    ids2d = ids.reshape(_N_ROWS // _CHUNK, _CHUNK)

    @functools.partial(
        pl.kernel,
        out_type=jax.ShapeDtypeStruct((_NC, _NS, _SEG_PER_TILE, _D), jnp.float32),
        mesh=mesh,
        scratch_types=[
            pltpu.VMEM((_CHUNK, _D), jnp.float32),
            pltpu.VMEM((_CHUNK,), jnp.int32),
            pltpu.VMEM_SHARED((_NUM_SEG, _D), jnp.float32),
        ],
    )
    def seg_sum(x_hbm, ids_hbm, zero_hbm, out_hbm, rows_v, idx_v, acc_sh):
        c = lax.axis_index("c")
        s = lax.axis_index("s")
        wid = c * _NS + s
        pltpu.sync_copy(zero_hbm, acc_sh.at[pl.ds(s * _SEG_PER_TILE, _SEG_PER_TILE)])
        plsc.subcore_barrier()

        def body(i, carry):
            pltpu.sync_copy(
                x_hbm.at[pl.ds((wid * _N_CHUNKS + i) * _CHUNK, _CHUNK)], rows_v)
            pltpu.sync_copy(ids_hbm.at[wid * _N_CHUNKS + i], idx_v)
            pltpu.sync_copy(rows_v, acc_sh.at[idx_v], add=True)
            return carry

        lax.fori_loop(0, _N_CHUNKS, body, 0)
        plsc.subcore_barrier()
        pltpu.sync_copy(
            acc_sh.at[pl.ds(s * _SEG_PER_TILE, _SEG_PER_TILE)],
            out_hbm.at[c, s])

    return seg_sum(x, ids2d, zeros).reshape(_NC, _NUM_SEG, _D)


def _heads_body(p_ref, w1a_ref, b1a_ref, w2a_ref, b2a_ref,
                w1c_ref, b1c_ref, w2c_ref, b2c_ref, lp_ref, v_ref):
    emb = p_ref[0] + p_ref[1]
    ha = jnp.tanh(
        jnp.dot(emb, w1a_ref[...], preferred_element_type=jnp.float32)
        + b1a_ref[...])
    logits = (jnp.dot(ha, w2a_ref[...], preferred_element_type=jnp.float32)
              + b2a_ref[...])
    m = jnp.max(logits, axis=-1, keepdims=True)
    lse = jnp.log(jnp.sum(jnp.exp(logits - m), axis=-1, keepdims=True)) + m
    lp_ref[...] = logits - lse
    hc = jnp.tanh(
        jnp.dot(emb, w1c_ref[...], preferred_element_type=jnp.float32)
        + b1c_ref[...])
    v_ref[...] = (jnp.dot(hc, w2c_ref[...], preferred_element_type=jnp.float32)
                  + b2c_ref[...])


def _tc_heads(partial, W1a, b1a, W2a, b2a, W1c, b1c, W2c, b2c):
    blk = 1000
    grid = _NUM_SEG // blk
    full = lambda shape: pl.BlockSpec(shape, lambda i: tuple(0 for _ in shape))
    return pl.pallas_call(
        _heads_body,
        grid=(grid,),
        in_specs=[
            pl.BlockSpec((_NC, blk, _D), lambda i: (0, i, 0)),
            full((_D, 64)), full((1, 64)),
            full((64, 7)), full((1, 7)),
            full((_D, 64)), full((1, 64)),
            full((64, 1)), full((1, 1)),
        ],
        out_specs=[
            pl.BlockSpec((blk, 7), lambda i: (i, 0)),
            pl.BlockSpec((blk, 1), lambda i: (i, 0)),
        ],
        out_shape=[
            jax.ShapeDtypeStruct((_NUM_SEG, 7), jnp.float32),
            jax.ShapeDtypeStruct((_NUM_SEG, 1), jnp.float32),
        ],
    )(partial, W1a, b1a.reshape(1, 64), W2a, b2a.reshape(1, 7),
      W1c, b1c.reshape(1, 64), W2c, b2c.reshape(1, 1))


@jax.jit
def kernel(x, m_batch, W1a, b1a, W2a, b2a, W1c, b1c, W2c, b2c):
    ids = m_batch.astype(jnp.int32)
    partial = _sc_segment_sum(x, ids)
    lp, v = _tc_heads(partial, W1a, b1a, W2a, b2a, W1c, b1c, W2c, b2c)
    return (lp, v)


# speedup vs baseline: 5.2089x; 1.5204x over previous
import functools

import jax
import jax.numpy as jnp
from jax import lax
from jax.experimental import pallas as pl
from jax.experimental.pallas import tpu as pltpu
from jax.experimental.pallas import tpu_sc as plsc

_NUM_SEG = 10000
_N_ROWS = 320000
_D = 128
_NC = 2
_NS = 16
_NW = _NC * _NS
_ROWS_PER_W = _N_ROWS // _NW
_CHUNK = 80
_N_CHUNKS = _ROWS_PER_W // _CHUNK
_SEG_PER_TILE = _NUM_SEG // _NS


def _sc_segment_sum(x, ids):
    mesh = plsc.VectorSubcoreMesh(core_axis_name="c", subcore_axis_name="s")
    zeros = jnp.zeros((_SEG_PER_TILE, _D), jnp.float32)
    ids3d = ids.reshape(_NW, _N_CHUNKS, _CHUNK)

    @functools.partial(
        pl.kernel,
        out_type=jax.ShapeDtypeStruct((_NC, _NS, _SEG_PER_TILE, _D), jnp.float32),
        mesh=mesh,
        scratch_types=[
            pltpu.VMEM((2, _CHUNK, _D), jnp.float32),
            pltpu.VMEM((_N_CHUNKS, _CHUNK), jnp.int32),
            pltpu.VMEM_SHARED((_NUM_SEG, _D), jnp.float32),
            pltpu.SemaphoreType.DMA,
            pltpu.SemaphoreType.DMA,
            pltpu.SemaphoreType.DMA,
            pltpu.SemaphoreType.DMA,
        ],
    )
    def seg_sum(x_hbm, ids_hbm, zero_hbm, out_hbm, bufs, idx_v, acc_sh,
                lsem0, lsem1, ssem0, ssem1):
        c = lax.axis_index("c")
        s = lax.axis_index("s")
        wid = c * _NS + s
        lsems = (lsem0, lsem1)
        ssems = (ssem0, ssem1)
        pltpu.sync_copy(ids_hbm.at[wid], idx_v)
        pltpu.sync_copy(zero_hbm, acc_sh.at[pl.ds(s * _SEG_PER_TILE, _SEG_PER_TILE)])
        plsc.subcore_barrier()

        base_row = wid * _ROWS_PER_W
        pltpu.async_copy(
            x_hbm.at[pl.ds(base_row, _CHUNK)], bufs.at[0], lsem0)

        def chunk_step(i, b):
            ob = 1 - b
            pltpu.make_async_copy(
                x_hbm.at[pl.ds(base_row + i * _CHUNK, _CHUNK)],
                bufs.at[b], lsems[b]).wait()
            pltpu.async_copy(bufs.at[b], acc_sh.at[idx_v.at[i]],
                             ssems[b], add=True)

            @pl.when(i >= 1)
            def _():
                pltpu.make_async_copy(
                    bufs.at[ob], acc_sh.at[idx_v.at[i - 1]],
                    ssems[ob]).wait()

            @pl.when(i + 1 < _N_CHUNKS)
            def _():
                pltpu.async_copy(
                    x_hbm.at[pl.ds(base_row + (i + 1) * _CHUNK, _CHUNK)],
                    bufs.at[ob], lsems[ob])

        def body(j, carry):
            chunk_step(2 * j, 0)
            chunk_step(2 * j + 1, 1)
            return carry

        lax.fori_loop(0, (_N_CHUNKS - 1) // 2, body, 0)
        last = _N_CHUNKS - 1
        pltpu.make_async_copy(
            x_hbm.at[pl.ds(base_row + last * _CHUNK, _CHUNK)],
            bufs.at[0], lsem0).wait()
        pltpu.async_copy(bufs.at[0], acc_sh.at[idx_v.at[last]],
                         ssem0, add=True)
        pltpu.make_async_copy(bufs.at[1], acc_sh.at[idx_v.at[last - 1]],
                              ssem1).wait()
        pltpu.make_async_copy(bufs.at[0], acc_sh.at[idx_v.at[last]],
                              ssem0).wait()
        plsc.subcore_barrier()
        pltpu.sync_copy(
            acc_sh.at[pl.ds(s * _SEG_PER_TILE, _SEG_PER_TILE)],
            out_hbm.at[c, s])

    return seg_sum(x, ids3d, zeros).reshape(_NC, _NUM_SEG, _D)


def _heads_body(p_ref, w1a_ref, b1a_ref, w2a_ref, b2a_ref,
                w1c_ref, b1c_ref, w2c_ref, b2c_ref, lp_ref, v_ref):
    emb = p_ref[0] + p_ref[1]
    ha = jnp.tanh(
        jnp.dot(emb, w1a_ref[...], preferred_element_type=jnp.float32)
        + b1a_ref[...])
    logits = (jnp.dot(ha, w2a_ref[...], preferred_element_type=jnp.float32)
              + b2a_ref[...])
    m = jnp.max(logits, axis=-1, keepdims=True)
    lse = jnp.log(jnp.sum(jnp.exp(logits - m), axis=-1, keepdims=True)) + m
    lp_ref[...] = logits - lse
    hc = jnp.tanh(
        jnp.dot(emb, w1c_ref[...], preferred_element_type=jnp.float32)
        + b1c_ref[...])
    v_ref[...] = (jnp.dot(hc, w2c_ref[...], preferred_element_type=jnp.float32)
                  + b2c_ref[...])


def _tc_heads(partial, W1a, b1a, W2a, b2a, W1c, b1c, W2c, b2c):
    blk = 1000
    grid = _NUM_SEG // blk
    full = lambda shape: pl.BlockSpec(shape, lambda i: tuple(0 for _ in shape))
    return pl.pallas_call(
        _heads_body,
        grid=(grid,),
        in_specs=[
            pl.BlockSpec((_NC, blk, _D), lambda i: (0, i, 0)),
            full((_D, 64)), full((1, 64)),
            full((64, 7)), full((1, 7)),
            full((_D, 64)), full((1, 64)),
            full((64, 1)), full((1, 1)),
        ],
        out_specs=[
            pl.BlockSpec((blk, 7), lambda i: (i, 0)),
            pl.BlockSpec((blk, 1), lambda i: (i, 0)),
        ],
        out_shape=[
            jax.ShapeDtypeStruct((_NUM_SEG, 7), jnp.float32),
            jax.ShapeDtypeStruct((_NUM_SEG, 1), jnp.float32),
        ],
    )(partial, W1a, b1a.reshape(1, 64), W2a, b2a.reshape(1, 7),
      W1c, b1c.reshape(1, 64), W2c, b2c.reshape(1, 1))


@jax.jit
def kernel(x, m_batch, W1a, b1a, W2a, b2a, W1c, b1c, W2c, b2c):
    ids = m_batch.astype(jnp.int32)
    partial = _sc_segment_sum(x, ids)
    lp, v = _tc_heads(partial, W1a, b1a, W2a, b2a, W1c, b1c, W2c, b2c)
    return (lp, v)


# speedup vs baseline: 6.3195x; 1.2132x over previous
import functools

import jax
import jax.numpy as jnp
from jax import lax
from jax.experimental import pallas as pl
from jax.experimental.pallas import tpu as pltpu
from jax.experimental.pallas import tpu_sc as plsc

_NUM_SEG = 10000
_N_ROWS = 320000
_D = 128
_NC = 2
_NS = 16
_NW = _NC * _NS
_ROWS_PER_W = _N_ROWS // _NW
_CHUNK = 80
_N_CHUNKS = _ROWS_PER_W // _CHUNK
_SEG_PER_TILE = _NUM_SEG // _NS
_NBUF = 3


def _sc_segment_sum(x, ids):
    mesh = plsc.VectorSubcoreMesh(core_axis_name="c", subcore_axis_name="s")
    zeros = jnp.zeros((_SEG_PER_TILE, _D), jnp.float32)
    ids3d = ids.reshape(_NW, _N_CHUNKS, _CHUNK)

    @functools.partial(
        pl.kernel,
        out_type=jax.ShapeDtypeStruct((_NC, _NS, _SEG_PER_TILE, _D), jnp.float32),
        mesh=mesh,
        scratch_types=[
            pltpu.VMEM((_NBUF, _CHUNK, _D), jnp.float32),
            pltpu.VMEM((_N_CHUNKS, _CHUNK), jnp.int32),
            pltpu.VMEM_SHARED((_NUM_SEG, _D), jnp.float32),
        ] + [pltpu.SemaphoreType.DMA] * (2 * _NBUF),
    )
    def seg_sum(x_hbm, ids_hbm, zero_hbm, out_hbm, bufs, idx_v, acc_sh,
                *sems):
        c = lax.axis_index("c")
        s = lax.axis_index("s")
        wid = c * _NS + s
        lsems = sems[:_NBUF]
        ssems = sems[_NBUF:]
        pltpu.sync_copy(ids_hbm.at[wid], idx_v)
        pltpu.sync_copy(zero_hbm, acc_sh.at[pl.ds(s * _SEG_PER_TILE, _SEG_PER_TILE)])
        plsc.subcore_barrier()

        base_row = wid * _ROWS_PER_W

        def start_load(i, b):
            pltpu.async_copy(
                x_hbm.at[pl.ds(base_row + i * _CHUNK, _CHUNK)],
                bufs.at[b], lsems[b])

        def wait_load(i, b):
            pltpu.make_async_copy(
                x_hbm.at[pl.ds(base_row + i * _CHUNK, _CHUNK)],
                bufs.at[b], lsems[b]).wait()

        def start_scatter(i, b):
            pltpu.async_copy(bufs.at[b], acc_sh.at[idx_v.at[i]],
                             ssems[b], add=True)

        def wait_scatter(i, b):
            pltpu.make_async_copy(bufs.at[b], acc_sh.at[idx_v.at[i]],
                                  ssems[b]).wait()

        for b in range(_NBUF - 1):
            start_load(b, b)

        def chunk_step(i, b):
            pb = (b + _NBUF - 1) % _NBUF
            wait_load(i, b)
            start_scatter(i, b)
            if isinstance(i, int):
                if i >= 1:
                    wait_scatter(i - 1, pb)
                if i + _NBUF - 1 < _N_CHUNKS:
                    start_load(i + _NBUF - 1, pb)
            else:
                @pl.when(i >= 1)
                def _():
                    wait_scatter(i - 1, pb)

                @pl.when(i + _NBUF - 1 < _N_CHUNKS)
                def _():
                    start_load(i + _NBUF - 1, pb)

        def body(j, carry):
            for b in range(_NBUF):
                chunk_step(_NBUF * j + b, b)
            return carry

        n_main = (_N_CHUNKS // _NBUF) * _NBUF
        lax.fori_loop(0, _N_CHUNKS // _NBUF, body, 0)
        for i in range(n_main, _N_CHUNKS):
            chunk_step(i, i % _NBUF)
        wait_scatter(_N_CHUNKS - 1, (_N_CHUNKS - 1) % _NBUF)
        plsc.subcore_barrier()
        pltpu.sync_copy(
            acc_sh.at[pl.ds(s * _SEG_PER_TILE, _SEG_PER_TILE)],
            out_hbm.at[c, s])

    return seg_sum(x, ids3d, zeros).reshape(_NC, _NUM_SEG, _D)


def _heads_body(p_ref, w1a_ref, b1a_ref, w2a_ref, b2a_ref,
                w1c_ref, b1c_ref, w2c_ref, b2c_ref, lp_ref, v_ref):
    emb = p_ref[0] + p_ref[1]
    ha = jnp.tanh(
        jnp.dot(emb, w1a_ref[...], preferred_element_type=jnp.float32)
        + b1a_ref[...])
    logits = (jnp.dot(ha, w2a_ref[...], preferred_element_type=jnp.float32)
              + b2a_ref[...])
    m = jnp.max(logits, axis=-1, keepdims=True)
    lse = jnp.log(jnp.sum(jnp.exp(logits - m), axis=-1, keepdims=True)) + m
    lp_ref[...] = logits - lse
    hc = jnp.tanh(
        jnp.dot(emb, w1c_ref[...], preferred_element_type=jnp.float32)
        + b1c_ref[...])
    v_ref[...] = (jnp.dot(hc, w2c_ref[...], preferred_element_type=jnp.float32)
                  + b2c_ref[...])


def _tc_heads(partial, W1a, b1a, W2a, b2a, W1c, b1c, W2c, b2c):
    blk = 1000
    grid = _NUM_SEG // blk
    full = lambda shape: pl.BlockSpec(shape, lambda i: tuple(0 for _ in shape))
    return pl.pallas_call(
        _heads_body,
        grid=(grid,),
        in_specs=[
            pl.BlockSpec((_NC, blk, _D), lambda i: (0, i, 0)),
            full((_D, 64)), full((1, 64)),
            full((64, 7)), full((1, 7)),
            full((_D, 64)), full((1, 64)),
            full((64, 1)), full((1, 1)),
        ],
        out_specs=[
            pl.BlockSpec((blk, 7), lambda i: (i, 0)),
            pl.BlockSpec((blk, 1), lambda i: (i, 0)),
        ],
        out_shape=[
            jax.ShapeDtypeStruct((_NUM_SEG, 7), jnp.float32),
            jax.ShapeDtypeStruct((_NUM_SEG, 1), jnp.float32),
        ],
    )(partial, W1a, b1a.reshape(1, 64), W2a, b2a.reshape(1, 7),
      W1c, b1c.reshape(1, 64), W2c, b2c.reshape(1, 1))


@jax.jit
def kernel(x, m_batch, W1a, b1a, W2a, b2a, W1c, b1c, W2c, b2c):
    ids = m_batch.astype(jnp.int32)
    partial = _sc_segment_sum(x, ids)
    lp, v = _tc_heads(partial, W1a, b1a, W2a, b2a, W1c, b1c, W2c, b2c)
    return (lp, v)


# speedup vs baseline: 6.6858x; 1.0580x over previous
import functools

import jax
import jax.numpy as jnp
from jax import lax
from jax.experimental import pallas as pl
from jax.experimental.pallas import tpu as pltpu
from jax.experimental.pallas import tpu_sc as plsc

_NUM_SEG = 10000
_N_ROWS = 320000
_D = 128
_NC = 2
_NS = 16
_NW = _NC * _NS
_ROWS_PER_W = _N_ROWS // _NW
_CHUNK = 80
_N_CHUNKS = _ROWS_PER_W // _CHUNK
_SEG_PER_TILE = _NUM_SEG // _NS
_NBUF = 4


def _sc_segment_sum(x, ids):
    mesh = plsc.VectorSubcoreMesh(core_axis_name="c", subcore_axis_name="s")
    zeros = jnp.zeros((_SEG_PER_TILE, _D), jnp.float32)
    ids3d = ids.reshape(_NW, _N_CHUNKS, 1, _CHUNK)

    @functools.partial(
        pl.kernel,
        out_type=jax.ShapeDtypeStruct((_NC, _NS, _SEG_PER_TILE, _D), jnp.float32),
        mesh=mesh,
        scratch_types=[
            pltpu.VMEM((_NBUF, _CHUNK, _D), jnp.float32),
            pltpu.VMEM((_NBUF, 1, _CHUNK), jnp.int32),
            pltpu.VMEM_SHARED((_NUM_SEG, _D), jnp.float32),
        ] + [pltpu.SemaphoreType.DMA] * (3 * _NBUF),
    )
    def seg_sum(x_hbm, ids_hbm, zero_hbm, out_hbm, bufs, idx_v, acc_sh,
                *sems):
        c = lax.axis_index("c")
        s = lax.axis_index("s")
        wid = c * _NS + s
        lsems = sems[:_NBUF]
        isems = sems[_NBUF:2 * _NBUF]
        ssems = sems[2 * _NBUF:]
        pltpu.sync_copy(zero_hbm, acc_sh.at[pl.ds(s * _SEG_PER_TILE, _SEG_PER_TILE)])
        plsc.subcore_barrier()

        base_row = wid * _ROWS_PER_W

        def start_load(i, b):
            pltpu.async_copy(
                x_hbm.at[pl.ds(base_row + i * _CHUNK, _CHUNK)],
                bufs.at[b], lsems[b])
            pltpu.async_copy(ids_hbm.at[wid, i], idx_v.at[b], isems[b])

        def wait_load(i, b):
            pltpu.make_async_copy(
                x_hbm.at[pl.ds(base_row + i * _CHUNK, _CHUNK)],
                bufs.at[b], lsems[b]).wait()
            pltpu.make_async_copy(ids_hbm.at[wid, i], idx_v.at[b],
                                  isems[b]).wait()

        def start_scatter(i, b):
            pltpu.async_copy(bufs.at[b], acc_sh.at[idx_v.at[b, 0]],
                             ssems[b], add=True)

        def wait_scatter(i, b):
            pltpu.make_async_copy(bufs.at[b], acc_sh.at[idx_v.at[b, 0]],
                                  ssems[b]).wait()

        for b in range(_NBUF - 1):
            start_load(b, b)

        def chunk_step(i, b):
            pb = (b + _NBUF - 1) % _NBUF
            wait_load(i, b)
            start_scatter(i, b)
            if isinstance(i, int):
                if i >= 1:
                    wait_scatter(i - 1, pb)
                if i + _NBUF - 1 < _N_CHUNKS:
                    start_load(i + _NBUF - 1, pb)
            else:
                @pl.when(i >= 1)
                def _():
                    wait_scatter(i - 1, pb)

                @pl.when(i + _NBUF - 1 < _N_CHUNKS)
                def _():
                    start_load(i + _NBUF - 1, pb)

        def body(j, carry):
            for b in range(_NBUF):
                chunk_step(_NBUF * j + b, b)
            return carry

        n_main = (_N_CHUNKS // _NBUF) * _NBUF
        lax.fori_loop(0, _N_CHUNKS // _NBUF, body, 0)
        for i in range(n_main, _N_CHUNKS):
            chunk_step(i, i % _NBUF)
        wait_scatter(_N_CHUNKS - 1, (_N_CHUNKS - 1) % _NBUF)
        plsc.subcore_barrier()
        pltpu.sync_copy(
            acc_sh.at[pl.ds(s * _SEG_PER_TILE, _SEG_PER_TILE)],
            out_hbm.at[c, s])

    return seg_sum(x, ids3d, zeros).reshape(_NC, _NUM_SEG, _D)


def _heads_body(p_ref, w1a_ref, b1a_ref, w2a_ref, b2a_ref,
                w1c_ref, b1c_ref, w2c_ref, b2c_ref, lp_ref, v_ref):
    emb = p_ref[0] + p_ref[1]
    ha = jnp.tanh(
        jnp.dot(emb, w1a_ref[...], preferred_element_type=jnp.float32)
        + b1a_ref[...])
    logits = (jnp.dot(ha, w2a_ref[...], preferred_element_type=jnp.float32)
              + b2a_ref[...])
    m = jnp.max(logits, axis=-1, keepdims=True)
    lse = jnp.log(jnp.sum(jnp.exp(logits - m), axis=-1, keepdims=True)) + m
    lp_ref[...] = logits - lse
    hc = jnp.tanh(
        jnp.dot(emb, w1c_ref[...], preferred_element_type=jnp.float32)
        + b1c_ref[...])
    v_ref[...] = (jnp.dot(hc, w2c_ref[...], preferred_element_type=jnp.float32)
                  + b2c_ref[...])


def _tc_heads(partial, W1a, b1a, W2a, b2a, W1c, b1c, W2c, b2c):
    blk = 1000
    grid = _NUM_SEG // blk
    full = lambda shape: pl.BlockSpec(shape, lambda i: tuple(0 for _ in shape))
    return pl.pallas_call(
        _heads_body,
        grid=(grid,),
        in_specs=[
            pl.BlockSpec((_NC, blk, _D), lambda i: (0, i, 0)),
            full((_D, 64)), full((1, 64)),
            full((64, 7)), full((1, 7)),
            full((_D, 64)), full((1, 64)),
            full((64, 1)), full((1, 1)),
        ],
        out_specs=[
            pl.BlockSpec((blk, 7), lambda i: (i, 0)),
            pl.BlockSpec((blk, 1), lambda i: (i, 0)),
        ],
        out_shape=[
            jax.ShapeDtypeStruct((_NUM_SEG, 7), jnp.float32),
            jax.ShapeDtypeStruct((_NUM_SEG, 1), jnp.float32),
        ],
    )(partial, W1a, b1a.reshape(1, 64), W2a, b2a.reshape(1, 7),
      W1c, b1c.reshape(1, 64), W2c, b2c.reshape(1, 1))


@jax.jit
def kernel(x, m_batch, W1a, b1a, W2a, b2a, W1c, b1c, W2c, b2c):
    ids = m_batch.astype(jnp.int32)
    partial = _sc_segment_sum(x, ids)
    lp, v = _tc_heads(partial, W1a, b1a, W2a, b2a, W1c, b1c, W2c, b2c)
    return (lp, v)
